# Initial kernel scaffold; baseline (speedup 1.0000x reference)
import functools

import jax
import jax.numpy as jnp
from jax import lax
from jax.experimental import pallas as pl
from jax.experimental.pallas import tpu as pltpu
from jax.experimental.pallas import tpu_sc as plsc

D = 64
NC = 2
NS = 16
NW = NC * NS
CHUNK = 128


@functools.partial(jax.jit, static_argnums=(2,))
def _sc_gather(table, idx3, nchunks):
    mesh = plsc.VectorSubcoreMesh(core_axis_name="c", subcore_axis_name="s")

    @functools.partial(
        pl.kernel,
        mesh=mesh,
        out_type=jax.ShapeDtypeStruct((NW, nchunks, CHUNK, D), jnp.float32),
        scratch_types=[
            pltpu.VMEM((nchunks, CHUNK), jnp.int32),
            pltpu.VMEM((CHUNK, D), jnp.float32),
            pltpu.SemaphoreType.DMA,
        ],
    )
    def k(table_hbm, idx_hbm, out_hbm, idx_v, rows_v, sem):
        w = lax.axis_index("s") * NC + lax.axis_index("c")
        pltpu.sync_copy(idx_hbm.at[w], idx_v)

        def body(j, carry):
            pltpu.async_copy(table_hbm.at[idx_v.at[j]], rows_v, sem).wait()
            pltpu.sync_copy(rows_v, out_hbm.at[w].at[j])
            return carry

        lax.fori_loop(0, nchunks, body, 0)

    return k(table, idx3)


def kernel(input_pos, pos_enc_table):
    b, s = input_pos.shape
    n = b * s
    idx3 = input_pos.reshape(NW, n // (NW * CHUNK), CHUNK).astype(jnp.int32)
    out = _sc_gather(pos_enc_table, idx3, n // (NW * CHUNK))
    return out.reshape(b, s, D)


# speedup vs baseline: 15.6205x; 15.6205x over previous
import functools

import jax
import jax.numpy as jnp
from jax import lax
from jax.experimental import pallas as pl
from jax.experimental.pallas import tpu as pltpu
from jax.experimental.pallas import tpu_sc as plsc

D = 64
NC = 2
NS = 16
NW = NC * NS
CHUNK = 128


@functools.partial(jax.jit, static_argnums=(2,))
def _sc_gather(table, idx3, nchunks):
    mesh = plsc.VectorSubcoreMesh(core_axis_name="c", subcore_axis_name="s")

    @functools.partial(
        pl.kernel,
        mesh=mesh,
        out_type=jax.ShapeDtypeStruct((NW, nchunks, CHUNK, D), jnp.float32),
        scratch_types=[
            pltpu.VMEM((nchunks, CHUNK), jnp.int32),
            pltpu.VMEM((CHUNK, D), jnp.float32),
            pltpu.SemaphoreType.DMA,
        ],
        compiler_params=pltpu.CompilerParams(use_tc_tiling_on_sc=False),
    )
    def k(table_hbm, idx_hbm, out_hbm, idx_v, rows_v, sem):
        w = lax.axis_index("s") * NC + lax.axis_index("c")
        pltpu.sync_copy(idx_hbm.at[w], idx_v)

        def body(j, carry):
            pltpu.async_copy(table_hbm.at[idx_v.at[j]], rows_v, sem).wait()
            pltpu.sync_copy(rows_v, out_hbm.at[w].at[j])
            return carry

        lax.fori_loop(0, nchunks, body, 0)

    return k(table, idx3)


def kernel(input_pos, pos_enc_table):
    b, s = input_pos.shape
    n = b * s
    idx3 = input_pos.reshape(NW, n // (NW * CHUNK), CHUNK).astype(jnp.int32)
    out = _sc_gather(pos_enc_table, idx3, n // (NW * CHUNK))
    return out.reshape(b, s, D)


# speedup vs baseline: 18.4910x; 1.1838x over previous
import functools

import jax
import jax.numpy as jnp
from jax import lax
from jax.experimental import pallas as pl
from jax.experimental.pallas import tpu as pltpu
from jax.experimental.pallas import tpu_sc as plsc

D = 64
NC = 2
NS = 16
NW = NC * NS
CHUNK = 128
NBUF = 8
K = 4


@functools.partial(jax.jit, static_argnums=(2,))
def _sc_gather(table, idx3, nchunks):
    mesh = plsc.VectorSubcoreMesh(core_axis_name="c", subcore_axis_name="s")

    @functools.partial(
        pl.kernel,
        mesh=mesh,
        out_type=jax.ShapeDtypeStruct((NW, nchunks, CHUNK, D), jnp.float32),
        scratch_types=[
            pltpu.VMEM((nchunks, CHUNK), jnp.int32),
            pltpu.VMEM((NBUF, CHUNK, D), jnp.float32),
            pltpu.SemaphoreType.DMA((NBUF,)),
            pltpu.SemaphoreType.DMA((NBUF,)),
        ],
        compiler_params=pltpu.CompilerParams(use_tc_tiling_on_sc=False),
    )
    def k(table_hbm, idx_hbm, out_hbm, idx_v, rows_v, gsem, ssem):
        w = lax.axis_index("s") * NC + lax.axis_index("c")
        pltpu.sync_copy(idx_hbm.at[w], idx_v)
        out_w = out_hbm.at[w]

        def gather_start(j, b):
            pltpu.async_copy(table_hbm.at[idx_v.at[j]], rows_v.at[b],
                             gsem.at[b])

        def gather_wait(j, b):
            pltpu.make_async_copy(table_hbm.at[idx_v.at[j]], rows_v.at[b],
                                  gsem.at[b]).wait()

        def store_start(j, b):
            pltpu.async_copy(rows_v.at[b], out_w.at[j], ssem.at[b])

        def store_wait(j, b):
            pltpu.make_async_copy(rows_v.at[b], out_w.at[j],
                                  ssem.at[b]).wait()

        def step(j, b, jn_valid, reuse):
            bg = (b + K) % NBUF
            if jn_valid:
                if reuse:
                    store_wait(j + K - NBUF, bg)
                gather_start(j + K, bg)
            gather_wait(j, b)
            store_start(j, b)

        for x in range(K):
            gather_start(x, x)

        for j in range(NBUF):
            step(j, j % NBUF, True, j + K >= NBUF)

        def body(blk, carry):
            j0 = blk * NBUF
            for b in range(NBUF):
                step(j0 + b, b, True, True)
            return carry

        lax.fori_loop(1, (nchunks - K) // NBUF, body, 0)

        for j in range(((nchunks - K) // NBUF) * NBUF, nchunks):
            step(j, j % NBUF, j + K < nchunks, True)

        for b in range(NBUF):
            store_wait(nchunks - NBUF + b, b)

    return k(table, idx3)


def kernel(input_pos, pos_enc_table):
    b, s = input_pos.shape
    n = b * s
    idx3 = input_pos.reshape(NW, n // (NW * CHUNK), CHUNK).astype(jnp.int32)
    out = _sc_gather(pos_enc_table, idx3, n // (NW * CHUNK))
    return out.reshape(b, s, D)
